# baseline (speedup 1.0000x reference)
import jax
import jax.numpy as jnp
from jax.experimental import pallas as pl

_L = 2048
_BLK = 1024


def _codes_body(seq_ref, rowmask_ref, colbit_ref):
    seq = seq_ref[...]
    m = jnp.max(seq, axis=0, keepdims=True)
    a = seq[0:1, :] == m
    c = (seq[1:2, :] == m) & ~a
    g = (seq[2:3, :] == m) & ~a & ~c
    degen = m < 1.0
    allowed = jnp.where(a, 8, jnp.where(c, 4, jnp.where(g, 10, 5)))
    rowmask_ref[...] = jnp.where(degen, 15, allowed) + 16
    colbit_ref[...] = jnp.where(
        degen, 16, jnp.where(a, 1, jnp.where(c, 2, jnp.where(g, 4, 8))))


def _mask_body(con_ref, rowmask_ref, colbit_ref, out_ref):
    keep = (rowmask_ref[...] & colbit_ref[...]) != 0
    out_ref[...] = jnp.where(keep, con_ref[...], 0.0)


def kernel(con, feat):
    con2 = con.reshape(_L, _L)
    seq = feat[0, :4, :, 0]
    rowmask, colbit = pl.pallas_call(
        _codes_body,
        out_shape=[
            jax.ShapeDtypeStruct((1, _L), jnp.int32),
            jax.ShapeDtypeStruct((1, _L), jnp.int32),
        ],
    )(seq)
    rowmask_col = rowmask.reshape(_L, 1)
    out = pl.pallas_call(
        _mask_body,
        grid=(_L // _BLK,),
        in_specs=[
            pl.BlockSpec((_BLK, _L), lambda i: (i, 0)),
            pl.BlockSpec((_BLK, 1), lambda i: (i, 0)),
            pl.BlockSpec((1, _L), lambda i: (0, 0)),
        ],
        out_specs=pl.BlockSpec((_BLK, _L), lambda i: (i, 0)),
        out_shape=jax.ShapeDtypeStruct((_L, _L), jnp.float32),
    )(con2, rowmask_col, colbit)
    return out.reshape(con.shape)
